# baseline (speedup 1.0000x reference)
import functools

import jax
import jax.numpy as jnp
from jax import lax
from jax.experimental import pallas as pl
from jax.experimental.pallas import tpu as pltpu
from jax.experimental.pallas import tpu_sc as plsc

BATCH = 16384


@functools.lru_cache(maxsize=1)
def _build():
    info = plsc.get_sparse_core_info()
    nc, ns = info.num_cores, info.num_subcores
    nw = nc * ns
    bpw = BATCH // nw
    mesh = plsc.VectorSubcoreMesh(core_axis_name="c", subcore_axis_name="s")

    @functools.partial(
        pl.kernel,
        mesh=mesh,
        compiler_params=pltpu.CompilerParams(needs_layout_passes=False),
        out_type=jax.ShapeDtypeStruct((BATCH,), jnp.float32),
        scratch_types=[
            pltpu.VMEM((bpw,), jnp.float32),
            pltpu.SemaphoreType.DMA,
        ],
    )
    def k(idx_i_hbm, idx_j_hbm, flat_hbm, out_hbm, out_v, sem):
        wid = lax.axis_index("s") * nc + lax.axis_index("c")
        base = wid * bpw
        pltpu.async_copy(flat_hbm.at[pl.ds(base, bpw)], out_v, sem).wait()
        pltpu.sync_copy(out_v, out_hbm.at[pl.ds(base, bpw)])

    return k


def kernel(user_index_i, user_index_j, user_embedding):
    k = _build()
    flat = user_embedding.T.reshape(16 * 1000000)
    return k(user_index_i.astype(jnp.int32),
             user_index_j.astype(jnp.int32),
             flat)


# speedup vs baseline: 2.7239x; 2.7239x over previous
import functools

import jax
import jax.numpy as jnp
from jax import lax
from jax.experimental import pallas as pl
from jax.experimental.pallas import tpu as pltpu
from jax.experimental.pallas import tpu_sc as plsc

BATCH = 16384
D = 16
L = 16
RPT = 8
CHUNK = 128
NBLK = 1000000 // RPT


@functools.lru_cache(maxsize=1)
def _build():
    info = plsc.get_sparse_core_info()
    nc, ns = info.num_cores, info.num_subcores
    nw = nc * ns
    bpw = BATCH // nw
    nchunk = bpw // CHUNK
    mesh = plsc.VectorSubcoreMesh(core_axis_name="c", subcore_axis_name="s")

    @functools.partial(
        pl.kernel,
        mesh=mesh,
        compiler_params=pltpu.CompilerParams(
            needs_layout_passes=False, use_tc_tiling_on_sc=False),
        out_type=jax.ShapeDtypeStruct((BATCH,), jnp.float32),
        scratch_types=[
            pltpu.VMEM((nchunk, CHUNK), jnp.int32),
            pltpu.VMEM((nchunk, CHUNK), jnp.int32),
            pltpu.VMEM((nchunk, CHUNK), jnp.int32),
            pltpu.VMEM((nchunk, CHUNK), jnp.int32),
            pltpu.VMEM((CHUNK, RPT, D), jnp.float32),
            pltpu.VMEM((CHUNK, RPT, D), jnp.float32),
            pltpu.VMEM((CHUNK, RPT, D), jnp.float32),
            pltpu.VMEM((CHUNK, RPT, D), jnp.float32),
            pltpu.VMEM((bpw,), jnp.float32),
            pltpu.SemaphoreType.DMA,
            pltpu.SemaphoreType.DMA,
        ],
    )
    def k(idx_i_hbm, idx_j_hbm, table_hbm, out_hbm,
          idxi_v, idxj_v, bli_v, blj_v, ti0, ti1, tj0, tj1, out_v,
          sem_i, sem_j):
        tiles_i = (ti0, ti1)
        tiles_j = (tj0, tj1)
        wid = lax.axis_index("s") * nc + lax.axis_index("c")
        base = wid * bpw
        for c in range(nchunk):
            pltpu.sync_copy(idx_i_hbm.at[pl.ds(base + c * CHUNK, CHUNK)],
                            idxi_v.at[c])
            pltpu.sync_copy(idx_j_hbm.at[pl.ds(base + c * CHUNK, CHUNK)],
                            idxj_v.at[c])
        for c in range(nchunk):
            for g in range(CHUNK // L):
                s = pl.ds(g * L, L)
                bli_v.at[c][s] = lax.shift_right_logical(idxi_v.at[c][s], 3)
                blj_v.at[c][s] = lax.shift_right_logical(idxj_v.at[c][s], 3)

        def fire(c, slot):
            return (pltpu.async_copy(table_hbm.at[bli_v.at[c]],
                                     tiles_i[slot], sem_i),
                    pltpu.async_copy(table_hbm.at[blj_v.at[c]],
                                     tiles_j[slot], sem_j))

        inflight = fire(0, 0)
        for c in range(nchunk):
            nxt = fire(c + 1, (c + 1) % 2) if c + 1 < nchunk else None
            inflight[0].wait()
            inflight[1].wait()
            slot = c % 2
            for g in range(CHUNK // L):
                s = pl.ds(g * L, L)
                bvec = g * L + lax.iota(jnp.int32, L)
                ri = idxi_v.at[c][s] & 7
                rj = idxj_v.at[c][s] & 7
                acc = jnp.zeros((L,), jnp.float32)
                for kk in range(D):
                    col = jnp.full((L,), kk, jnp.int32)
                    a = plsc.load_gather(tiles_i[slot], [bvec, ri, col])
                    b = plsc.load_gather(tiles_j[slot], [bvec, rj, col])
                    acc = acc + a * b
                out_v[pl.ds(c * CHUNK + g * L, L)] = acc
            inflight = nxt
        pltpu.sync_copy(out_v, out_hbm.at[pl.ds(base, bpw)])

    return k


def kernel(user_index_i, user_index_j, user_embedding):
    k = _build()
    table_blocks = user_embedding.reshape(NBLK, RPT, D)
    return k(user_index_i.astype(jnp.int32),
             user_index_j.astype(jnp.int32),
             table_blocks)
